# baseline (speedup 1.0000x reference)
import functools

import jax
import jax.numpy as jnp
from jax import lax
from jax.experimental import pallas as pl
from jax.experimental.pallas import tpu as pltpu
from jax.experimental.pallas import tpu_sc as plsc

_N = 100000
_E = 3200000
_H1, _H2, _H3 = 64, 32, 16

_NPAD = 100352
_NT = _NPAD // 16
_EPAD = 3244032
_EB = 1024
_RB_DEG = _EPAD // 32 // _EB
_RB_ALL = _EPAD // 16 // _EB

_mesh = plsc.VectorSubcoreMesh(core_axis_name="c", subcore_axis_name="s")
_sc_params = pltpu.CompilerParams(
    needs_layout_passes=False, use_tc_tiling_on_sc=False)


def _zero_1d(buf, nwords):
    z = jnp.zeros((16,), jnp.float32)

    def st(i, carry):
        buf[pl.ds(i * 16, 16)] = z
        return carry

    lax.fori_loop(0, nwords // 16, st, 0)


def _zero_rows(buf, nrows):
    z = jnp.zeros((16,), jnp.float32)

    def st(i, carry):
        buf[i, :] = z
        return carry

    lax.fori_loop(0, nrows, st, 0)


@functools.partial(
    pl.kernel,
    out_type=jax.ShapeDtypeStruct((2, _NPAD), jnp.float32),
    mesh=_mesh,
    compiler_params=_sc_params,
    scratch_types=[
        pltpu.VMEM((3, _EB), jnp.int32),
        pltpu.VMEM((_EB,), jnp.float32),
        pltpu.VMEM((_NT,), jnp.float32),
        pltpu.VMEM_SHARED((_NPAD,), jnp.float32),
        pltpu.SemaphoreType.DMA,
        pltpu.SemaphoreType.DMA,
    ],
)
def _sc_deg(edges_hbm, out_hbm, idx_v, ones_v, zbuf_v, acc_sh, isem, ssem):
    c = lax.axis_index("c")
    s = lax.axis_index("s")
    _zero_1d(zbuf_v, _NT)
    one = jnp.ones((16,), jnp.float32)

    def st1(i, carry):
        ones_v[pl.ds(i * 16, 16)] = one
        return carry

    lax.fori_loop(0, _EB // 16, st1, 0)
    pltpu.sync_copy(zbuf_v, acc_sh.at[pl.ds(s * _NT, _NT)])
    plsc.subcore_barrier()

    base_e = (c * 16 + s) * (_RB_DEG * _EB)

    def load_idx(r, b):
        pltpu.async_copy(edges_hbm.at[1, pl.ds(base_e + r * _EB, _EB)],
                         idx_v.at[b], isem)

    def wait_idx(b):
        pltpu.make_async_copy(edges_hbm.at[1, pl.ds(0, _EB)],
                              idx_v.at[b], isem).wait()

    def wait_s(b):
        pltpu.make_async_copy(ones_v, acc_sh.at[idx_v.at[b]], ssem).wait()

    load_idx(0, 0)

    def half(r, b):
        bn = (b + 1) % 3

        @pl.when(r >= 2)
        def _():
            wait_s(bn)

        @pl.when(r + 1 < _RB_DEG)
        def _():
            load_idx(r + 1, bn)
        wait_idx(b)
        pltpu.async_copy(ones_v, acc_sh.at[idx_v.at[b]], ssem, add=True)

    def blk(r3, carry):
        half(r3 * 3, 0)
        half(r3 * 3 + 1, 1)
        half(r3 * 3 + 2, 2)
        return carry

    lax.fori_loop(0, _RB_DEG // 3, blk, 0)
    wait_s((_RB_DEG - 2) % 3)
    wait_s((_RB_DEG - 1) % 3)
    plsc.subcore_barrier()
    pltpu.sync_copy(acc_sh.at[pl.ds(s * _NT, _NT)], zbuf_v)
    pltpu.sync_copy(zbuf_v, out_hbm.at[c, pl.ds(s * _NT, _NT)])


@functools.partial(
    pl.kernel,
    out_type=jax.ShapeDtypeStruct((2, _NPAD), jnp.float32),
    mesh=_mesh,
    compiler_params=_sc_params,
    scratch_types=[
        pltpu.VMEM((_NPAD,), jnp.float32),
        pltpu.VMEM((3, _EB), jnp.int32),
        pltpu.VMEM((3, _EB), jnp.int32),
        pltpu.VMEM((3, _EB), jnp.float32),
        pltpu.VMEM((_NT,), jnp.float32),
        pltpu.VMEM_SHARED((_NPAD,), jnp.float32),
        pltpu.SemaphoreType.DMA,
        pltpu.SemaphoreType.DMA,
    ],
)
def _sc_tu(edges_hbm, tab_hbm, out_hbm, table_v, idxg_v, idxs_v, vals_v,
           zbuf_v, acc_sh, isem, ssem):
    c = lax.axis_index("c")
    s = lax.axis_index("s")
    _zero_1d(zbuf_v, _NT)
    pltpu.sync_copy(zbuf_v, acc_sh.at[pl.ds(s * _NT, _NT)])
    pltpu.sync_copy(tab_hbm.at[c], table_v)
    plsc.subcore_barrier()

    gcomp = c
    scomp = 1 - c
    base_e = s * (_RB_ALL * _EB)

    def load_idx(r, b):
        e0 = base_e + r * _EB
        pltpu.async_copy(edges_hbm.at[gcomp, pl.ds(e0, _EB)],
                         idxg_v.at[b], isem)
        pltpu.async_copy(edges_hbm.at[scomp, pl.ds(e0, _EB)],
                         idxs_v.at[b], isem)

    def wait_idx(b):
        pltpu.make_async_copy(edges_hbm.at[gcomp, pl.ds(0, _EB)],
                              idxg_v.at[b], isem).wait()
        pltpu.make_async_copy(edges_hbm.at[scomp, pl.ds(0, _EB)],
                              idxs_v.at[b], isem).wait()

    def wait_s(b):
        pltpu.make_async_copy(vals_v.at[b], acc_sh.at[idxs_v.at[b]],
                              ssem).wait()

    load_idx(0, 0)

    def half(r, b):
        bn = (b + 1) % 3

        @pl.when(r >= 2)
        def _():
            wait_s(bn)

        @pl.when(r + 1 < _RB_ALL)
        def _():
            load_idx(r + 1, bn)
        wait_idx(b)
        for q in range(_EB // 16):
            iv = idxg_v[b, pl.ds(q * 16, 16)]
            vals_v[b, pl.ds(q * 16, 16)] = plsc.load_gather(table_v, [iv])
        pltpu.async_copy(vals_v.at[b], acc_sh.at[idxs_v.at[b]], ssem,
                         add=True)

    def blk(r3, carry):
        half(r3 * 3, 0)
        half(r3 * 3 + 1, 1)
        half(r3 * 3 + 2, 2)
        return carry

    lax.fori_loop(0, _RB_ALL // 3, blk, 0)
    wait_s((_RB_ALL - 2) % 3)
    wait_s((_RB_ALL - 1) % 3)
    plsc.subcore_barrier()
    pltpu.sync_copy(acc_sh.at[pl.ds(s * _NT, _NT)], zbuf_v)
    pltpu.sync_copy(zbuf_v, out_hbm.at[c, pl.ds(s * _NT, _NT)])


_EB3 = 384
_RB3 = _EPAD // 32 // _EB3
_OB = _NT // 64


_NR = 4
_SG = _EB3 // 128


@functools.partial(
    pl.kernel,
    out_type=jax.ShapeDtypeStruct((2, _NPAD, 32), jnp.bfloat16),
    mesh=_mesh,
    compiler_params=_sc_params,
    scratch_types=[
        pltpu.VMEM((_NR, _SG, 128), jnp.int32),
        pltpu.VMEM((_NR, _SG, 128), jnp.int32),
        pltpu.VMEM((_NR, _SG, 128, 32), jnp.bfloat16),
        pltpu.VMEM((_OB, 32), jnp.bfloat16),
        pltpu.VMEM_SHARED((_NPAD, 32), jnp.bfloat16),
        pltpu.SemaphoreType.DMA,
        pltpu.SemaphoreType.DMA,
        pltpu.SemaphoreType.DMA,
    ],
)
def _sc_acc(edges_hbm, g_hbm, out_hbm, idxg_v, idxs_v, rows_v, obuf_v,
            acc_sh, isem, gsem, ssem):
    c = lax.axis_index("c")
    s = lax.axis_index("s")
    zrow = jnp.zeros((32,), jnp.bfloat16)

    def zr(i, carry):
        obuf_v[i, :] = zrow
        return carry

    lax.fori_loop(0, _OB, zr, 0)
    for k in range(64):
        pltpu.sync_copy(obuf_v, acc_sh.at[pl.ds(s * _NT + k * _OB, _OB)])
    plsc.subcore_barrier()

    base_e = (c * 16 + s) * (_RB3 * _EB3)

    def load_idx(r, b):
        e0 = base_e + r * _EB3
        for j in range(_SG):
            pltpu.async_copy(edges_hbm.at[0, pl.ds(e0 + j * 128, 128)],
                             idxg_v.at[b, j], isem)
            pltpu.async_copy(edges_hbm.at[1, pl.ds(e0 + j * 128, 128)],
                             idxs_v.at[b, j], isem)

    def wait_idx(b):
        for j in range(_SG):
            pltpu.make_async_copy(edges_hbm.at[0, pl.ds(0, 128)],
                                  idxg_v.at[b, j], isem).wait()
            pltpu.make_async_copy(edges_hbm.at[1, pl.ds(0, 128)],
                                  idxs_v.at[b, j], isem).wait()

    def fire_g(b):
        for j in range(_SG):
            pltpu.async_copy(g_hbm.at[idxg_v.at[b, j]],
                             rows_v.at[b, j], gsem)

    def wait_g(b):
        for j in range(_SG):
            pltpu.make_async_copy(g_hbm.at[idxg_v.at[b, j]],
                                  rows_v.at[b, j], gsem).wait()

    def fire_s(b):
        for j in range(_SG):
            pltpu.async_copy(rows_v.at[b, j], acc_sh.at[idxs_v.at[b, j]],
                             ssem, add=True)

    def wait_s(b):
        for j in range(_SG):
            pltpu.make_async_copy(rows_v.at[b, j],
                                  acc_sh.at[idxs_v.at[b, j]], ssem).wait()

    load_idx(0, 0)
    load_idx(1, 1)
    wait_idx(0)
    fire_g(0)

    def step(r, b):
        b1 = (b + 1) % _NR
        b2 = (b + 2) % _NR

        @pl.when(r >= 2)
        def _():
            wait_s(b2)

        @pl.when(r + 2 < _RB3)
        def _():
            load_idx(r + 2, b2)

        @pl.when(r + 1 < _RB3)
        def _():
            wait_idx(b1)
            fire_g(b1)
        wait_g(b)
        fire_s(b)

    def blk(r6, carry):
        for u in range(_NR):
            step(r6 * _NR + u, u)
        return carry

    lax.fori_loop(0, _RB3 // _NR, blk, 0)
    wait_s((_RB3 - 2) % _NR)
    wait_s((_RB3 - 1) % _NR)
    plsc.subcore_barrier()
    for k in range(64):
        pltpu.sync_copy(acc_sh.at[pl.ds(s * _NT + k * _OB, _OB)], obuf_v)
        pltpu.sync_copy(obuf_v, out_hbm.at[c, pl.ds(s * _NT + k * _OB, _OB)])


def _tc_k1_body(deg_ref, x_ref, tab_ref):
    deg = deg_ref[0:1, :] + deg_ref[1:2, :] + 1.0
    dinv = lax.rsqrt(deg)
    tab_ref[0:1, :] = x_ref[...] * dinv
    tab_ref[1:2, :] = dinv


def _tc_k1(deg2, x_row):
    return pl.pallas_call(
        _tc_k1_body,
        out_shape=jax.ShapeDtypeStruct((2, _NPAD), jnp.float32),
    )(deg2, x_row)


_BLK = _NT
_GRID = _NPAD // _BLK


def _tc_k2_body(t_ref, u_ref, x_ref, dinv_ref, m_ref, w1_ref, b1_ref, w2_ref,
                g_ref, c_ref):
    dinv = dinv_ref[...]
    s = dinv * t_ref[...] + dinv * dinv * x_ref[...]
    h1 = s * w1_ref[...] + b1_ref[...]
    h1 = jnp.where(h1 > 0, h1, 0.1 * h1)
    hw2 = jnp.dot(h1, w2_ref[...], preferred_element_type=jnp.float32)
    g_ref[...] = (dinv * hw2).astype(jnp.bfloat16)
    c_ref[...] = (dinv * u_ref[...] + dinv * dinv) * m_ref[...]


def _tc_k2(t_c, u_c, x_c, dinv_c, mask_c, W1, b1, W2):
    col = pl.BlockSpec((_BLK, 1), lambda i: (i, 0))
    full = lambda shape: pl.BlockSpec(shape, lambda i: tuple(0 for _ in shape))
    return pl.pallas_call(
        _tc_k2_body,
        grid=(_GRID,),
        in_specs=[col, col, col, col, col,
                  full((1, _H1)), full((1, _H1)), full((_H1, _H2))],
        out_specs=[pl.BlockSpec((_BLK, _H2), lambda i: (i, 0)), col],
        out_shape=[
            jax.ShapeDtypeStruct((_NPAD, _H2), jnp.bfloat16),
            jax.ShapeDtypeStruct((_NPAD, 1), jnp.float32),
        ],
    )(t_c, u_c, x_c, dinv_c, mask_c, W1, b1, W2)


def _tc_k3_body(acc_ref, g_ref, dinv_ref, c_ref, b2_ref, w3_ref, b3_ref,
                out_ref, racc):
    i = pl.program_id(0)
    a = (acc_ref[0].astype(jnp.float32) + acc_ref[1].astype(jnp.float32)
         + g_ref[...].astype(jnp.float32))
    h2 = dinv_ref[...] * a + b2_ref[...]
    h2 = jnp.where(h2 > 0, h2, 0.1 * h2)
    pr = jnp.sum(c_ref[...] * h2, axis=0, keepdims=True)

    @pl.when(i == 0)
    def _():
        racc[...] = jnp.zeros_like(racc)

    racc[0:1, 0:_H2] += pr
    out_ref[...] = (
        jnp.dot(racc[0:1, 0:_H2], w3_ref[...],
                preferred_element_type=jnp.float32) / _N + b3_ref[...])


def _tc_k3(acc3, g3, dinv_c, c_c, b2, W3, b3):
    col = pl.BlockSpec((_BLK, 1), lambda i: (i, 0))
    full = lambda shape: pl.BlockSpec(shape, lambda i: tuple(0 for _ in shape))
    return pl.pallas_call(
        _tc_k3_body,
        grid=(_GRID,),
        in_specs=[pl.BlockSpec((2, _BLK, _H2), lambda i: (0, i, 0)),
                  pl.BlockSpec((_BLK, _H2), lambda i: (i, 0)), col, col,
                  full((1, _H2)), full((_H2, _H3)), full((1, _H3))],
        out_specs=full((1, _H3)),
        out_shape=jax.ShapeDtypeStruct((1, _H3), jnp.float32),
        scratch_shapes=[pltpu.VMEM((1, _H2), jnp.float32)],
    )(acc3, g3, dinv_c, c_c, b2, W3, b3)


def kernel(x, edge_index, W1, b1, W2, b2, W3, b3):
    pad1 = _N + (jnp.arange(_EPAD - _E, dtype=jnp.int32) % (_NPAD - _N))
    pad = jnp.stack([pad1, pad1])
    edges = jnp.concatenate([edge_index, pad], axis=1)

    xs = jnp.concatenate([x[:, 0], jnp.zeros((_NPAD - _N,), jnp.float32)])
    x_row = xs.reshape(1, _NPAD)
    x_col = xs.reshape(_NPAD, 1)
    mask_c = (jnp.arange(_NPAD) < _N).astype(jnp.float32).reshape(_NPAD, 1)

    deg2 = _sc_deg(edges)
    tab2 = _tc_k1(deg2, x_row)
    tu2 = _sc_tu(edges, tab2)
    dinv_c = tab2[1].reshape(_NPAD, 1)
    t_c = tu2[0].reshape(_NPAD, 1)
    u_c = tu2[1].reshape(_NPAD, 1)
    g3, c_c = _tc_k2(t_c, u_c, x_col, dinv_c, mask_c,
                     W1, b1.reshape(1, _H1), W2)
    acc3 = _sc_acc(edges, g3)
    out = _tc_k3(acc3, g3, dinv_c, c_c, b2.reshape(1, _H2), W3,
                 b3.reshape(1, _H3))
    return out.reshape(_H3)


# speedup vs baseline: 1.0329x; 1.0329x over previous
import functools

import jax
import jax.numpy as jnp
from jax import lax
from jax.experimental import pallas as pl
from jax.experimental.pallas import tpu as pltpu
from jax.experimental.pallas import tpu_sc as plsc

_N = 100000
_E = 3200000
_H1, _H2, _H3 = 64, 32, 16

_NPAD = 100352
_NT = _NPAD // 16
_EPAD = 3244032
_EB = 1024
_RB_DEG = _EPAD // 32 // _EB
_RB_ALL = _EPAD // 16 // _EB

_mesh = plsc.VectorSubcoreMesh(core_axis_name="c", subcore_axis_name="s")
_sc_params = pltpu.CompilerParams(
    needs_layout_passes=False, use_tc_tiling_on_sc=False)


def _zero_1d(buf, nwords):
    z = jnp.zeros((16,), jnp.float32)

    def st(i, carry):
        buf[pl.ds(i * 16, 16)] = z
        return carry

    lax.fori_loop(0, nwords // 16, st, 0)


def _zero_rows(buf, nrows):
    z = jnp.zeros((16,), jnp.float32)

    def st(i, carry):
        buf[i, :] = z
        return carry

    lax.fori_loop(0, nrows, st, 0)


@functools.partial(
    pl.kernel,
    out_type=jax.ShapeDtypeStruct((2, _NPAD), jnp.float32),
    mesh=_mesh,
    compiler_params=_sc_params,
    scratch_types=[
        pltpu.VMEM((3, _EB), jnp.int32),
        pltpu.VMEM((_EB,), jnp.float32),
        pltpu.VMEM((_NT,), jnp.float32),
        pltpu.VMEM_SHARED((_NPAD,), jnp.float32),
        pltpu.SemaphoreType.DMA,
        pltpu.SemaphoreType.DMA,
    ],
)
def _sc_deg(edges_hbm, out_hbm, idx_v, ones_v, zbuf_v, acc_sh, isem, ssem):
    c = lax.axis_index("c")
    s = lax.axis_index("s")
    _zero_1d(zbuf_v, _NT)
    one = jnp.ones((16,), jnp.float32)

    def st1(i, carry):
        ones_v[pl.ds(i * 16, 16)] = one
        return carry

    lax.fori_loop(0, _EB // 16, st1, 0)
    pltpu.sync_copy(zbuf_v, acc_sh.at[pl.ds(s * _NT, _NT)])
    plsc.subcore_barrier()

    base_e = (c * 16 + s) * (_RB_DEG * _EB)

    def load_idx(r, b):
        pltpu.async_copy(edges_hbm.at[1, pl.ds(base_e + r * _EB, _EB)],
                         idx_v.at[b], isem)

    def wait_idx(b):
        pltpu.make_async_copy(edges_hbm.at[1, pl.ds(0, _EB)],
                              idx_v.at[b], isem).wait()

    def wait_s(b):
        pltpu.make_async_copy(ones_v, acc_sh.at[idx_v.at[b]], ssem).wait()

    load_idx(0, 0)

    def half(r, b):
        bn = (b + 1) % 3

        @pl.when(r >= 2)
        def _():
            wait_s(bn)

        @pl.when(r + 1 < _RB_DEG)
        def _():
            load_idx(r + 1, bn)
        wait_idx(b)
        pltpu.async_copy(ones_v, acc_sh.at[idx_v.at[b]], ssem, add=True)

    def blk(r3, carry):
        half(r3 * 3, 0)
        half(r3 * 3 + 1, 1)
        half(r3 * 3 + 2, 2)
        return carry

    lax.fori_loop(0, _RB_DEG // 3, blk, 0)
    wait_s((_RB_DEG - 2) % 3)
    wait_s((_RB_DEG - 1) % 3)
    plsc.subcore_barrier()
    pltpu.sync_copy(acc_sh.at[pl.ds(s * _NT, _NT)], zbuf_v)
    pltpu.sync_copy(zbuf_v, out_hbm.at[c, pl.ds(s * _NT, _NT)])


@functools.partial(
    pl.kernel,
    out_type=jax.ShapeDtypeStruct((2, _NPAD), jnp.float32),
    mesh=_mesh,
    compiler_params=_sc_params,
    scratch_types=[
        pltpu.VMEM((_NPAD,), jnp.float32),
        pltpu.VMEM((3, _EB), jnp.int32),
        pltpu.VMEM((3, _EB), jnp.int32),
        pltpu.VMEM((3, _EB), jnp.float32),
        pltpu.VMEM((_NT,), jnp.float32),
        pltpu.VMEM_SHARED((_NPAD,), jnp.float32),
        pltpu.SemaphoreType.DMA,
        pltpu.SemaphoreType.DMA,
    ],
)
def _sc_tu(edges_hbm, tab_hbm, out_hbm, table_v, idxg_v, idxs_v, vals_v,
           zbuf_v, acc_sh, isem, ssem):
    c = lax.axis_index("c")
    s = lax.axis_index("s")
    _zero_1d(zbuf_v, _NT)
    pltpu.sync_copy(zbuf_v, acc_sh.at[pl.ds(s * _NT, _NT)])
    pltpu.sync_copy(tab_hbm.at[c], table_v)
    plsc.subcore_barrier()

    gcomp = c
    scomp = 1 - c
    base_e = s * (_RB_ALL * _EB)

    def load_idx(r, b):
        e0 = base_e + r * _EB
        pltpu.async_copy(edges_hbm.at[gcomp, pl.ds(e0, _EB)],
                         idxg_v.at[b], isem)
        pltpu.async_copy(edges_hbm.at[scomp, pl.ds(e0, _EB)],
                         idxs_v.at[b], isem)

    def wait_idx(b):
        pltpu.make_async_copy(edges_hbm.at[gcomp, pl.ds(0, _EB)],
                              idxg_v.at[b], isem).wait()
        pltpu.make_async_copy(edges_hbm.at[scomp, pl.ds(0, _EB)],
                              idxs_v.at[b], isem).wait()

    def wait_s(b):
        pltpu.make_async_copy(vals_v.at[b], acc_sh.at[idxs_v.at[b]],
                              ssem).wait()

    load_idx(0, 0)

    def half(r, b):
        bn = (b + 1) % 3

        @pl.when(r >= 2)
        def _():
            wait_s(bn)

        @pl.when(r + 1 < _RB_ALL)
        def _():
            load_idx(r + 1, bn)
        wait_idx(b)
        for q in range(_EB // 16):
            iv = idxg_v[b, pl.ds(q * 16, 16)]
            vals_v[b, pl.ds(q * 16, 16)] = plsc.load_gather(table_v, [iv])
        pltpu.async_copy(vals_v.at[b], acc_sh.at[idxs_v.at[b]], ssem,
                         add=True)

    def blk(r3, carry):
        half(r3 * 3, 0)
        half(r3 * 3 + 1, 1)
        half(r3 * 3 + 2, 2)
        return carry

    lax.fori_loop(0, _RB_ALL // 3, blk, 0)
    wait_s((_RB_ALL - 2) % 3)
    wait_s((_RB_ALL - 1) % 3)
    plsc.subcore_barrier()
    pltpu.sync_copy(acc_sh.at[pl.ds(s * _NT, _NT)], zbuf_v)
    pltpu.sync_copy(zbuf_v, out_hbm.at[c, pl.ds(s * _NT, _NT)])


_EB3 = 256
_RB3 = _EPAD // 32 // _EB3
_OB = _NT // 64


_NR = 6
_SG = _EB3 // 128


@functools.partial(
    pl.kernel,
    out_type=jax.ShapeDtypeStruct((2, _NPAD, 32), jnp.bfloat16),
    mesh=_mesh,
    compiler_params=_sc_params,
    scratch_types=[
        pltpu.VMEM((_NR, _SG, 128), jnp.int32),
        pltpu.VMEM((_NR, _SG, 128), jnp.int32),
        pltpu.VMEM((_NR, _SG, 128, 32), jnp.bfloat16),
        pltpu.VMEM((_OB, 32), jnp.bfloat16),
        pltpu.VMEM_SHARED((_NPAD, 32), jnp.bfloat16),
        pltpu.SemaphoreType.DMA,
        pltpu.SemaphoreType.DMA,
        pltpu.SemaphoreType.DMA,
    ],
)
def _sc_acc(edges_hbm, g_hbm, out_hbm, idxg_v, idxs_v, rows_v, obuf_v,
            acc_sh, isem, gsem, ssem):
    c = lax.axis_index("c")
    s = lax.axis_index("s")
    zrow = jnp.zeros((32,), jnp.bfloat16)

    def zr(i, carry):
        obuf_v[i, :] = zrow
        return carry

    lax.fori_loop(0, _OB, zr, 0)
    for k in range(64):
        pltpu.sync_copy(obuf_v, acc_sh.at[pl.ds(s * _NT + k * _OB, _OB)])
    plsc.subcore_barrier()

    base_e = (c * 16 + s) * (_RB3 * _EB3)

    def load_idx(r, b):
        e0 = base_e + r * _EB3
        for j in range(_SG):
            pltpu.async_copy(edges_hbm.at[0, pl.ds(e0 + j * 128, 128)],
                             idxg_v.at[b, j], isem)
            pltpu.async_copy(edges_hbm.at[1, pl.ds(e0 + j * 128, 128)],
                             idxs_v.at[b, j], isem)

    def wait_idx(b):
        for j in range(_SG):
            pltpu.make_async_copy(edges_hbm.at[0, pl.ds(0, 128)],
                                  idxg_v.at[b, j], isem).wait()
            pltpu.make_async_copy(edges_hbm.at[1, pl.ds(0, 128)],
                                  idxs_v.at[b, j], isem).wait()

    def fire_g(b):
        for j in range(_SG):
            pltpu.async_copy(g_hbm.at[idxg_v.at[b, j]],
                             rows_v.at[b, j], gsem)

    def wait_g(b):
        for j in range(_SG):
            pltpu.make_async_copy(g_hbm.at[idxg_v.at[b, j]],
                                  rows_v.at[b, j], gsem).wait()

    def fire_s(b):
        for j in range(_SG):
            pltpu.async_copy(rows_v.at[b, j], acc_sh.at[idxs_v.at[b, j]],
                             ssem, add=True)

    def wait_s(b):
        for j in range(_SG):
            pltpu.make_async_copy(rows_v.at[b, j],
                                  acc_sh.at[idxs_v.at[b, j]], ssem).wait()

    load_idx(0, 0)
    load_idx(1, 1)
    load_idx(2, 2)
    wait_idx(0)
    fire_g(0)
    wait_idx(1)
    fire_g(1)

    def step(r, b):
        b2 = (b + 2) % _NR
        b3 = (b + 3) % _NR

        @pl.when(r >= 3)
        def _():
            wait_s(b3)

        @pl.when(r + 3 < _RB3)
        def _():
            load_idx(r + 3, b3)

        @pl.when(r + 2 < _RB3)
        def _():
            wait_idx(b2)
            fire_g(b2)
        wait_g(b)
        fire_s(b)

    def blk(r6, carry):
        for u in range(_NR):
            step(r6 * _NR + u, u)
        return carry

    lax.fori_loop(0, _RB3 // _NR, blk, 0)
    wait_s((_RB3 - 3) % _NR)
    wait_s((_RB3 - 2) % _NR)
    wait_s((_RB3 - 1) % _NR)
    plsc.subcore_barrier()
    for k in range(64):
        pltpu.sync_copy(acc_sh.at[pl.ds(s * _NT + k * _OB, _OB)], obuf_v)
        pltpu.sync_copy(obuf_v, out_hbm.at[c, pl.ds(s * _NT + k * _OB, _OB)])


def _tc_k1_body(deg_ref, x_ref, tab_ref):
    deg = deg_ref[0:1, :] + deg_ref[1:2, :] + 1.0
    dinv = lax.rsqrt(deg)
    tab_ref[0:1, :] = x_ref[...] * dinv
    tab_ref[1:2, :] = dinv


def _tc_k1(deg2, x_row):
    return pl.pallas_call(
        _tc_k1_body,
        out_shape=jax.ShapeDtypeStruct((2, _NPAD), jnp.float32),
    )(deg2, x_row)


_BLK = _NT
_GRID = _NPAD // _BLK


def _tc_k2_body(t_ref, u_ref, x_ref, dinv_ref, m_ref, w1_ref, b1_ref, w2_ref,
                g_ref, c_ref):
    dinv = dinv_ref[...]
    s = dinv * t_ref[...] + dinv * dinv * x_ref[...]
    h1 = s * w1_ref[...] + b1_ref[...]
    h1 = jnp.where(h1 > 0, h1, 0.1 * h1)
    hw2 = jnp.dot(h1, w2_ref[...], preferred_element_type=jnp.float32)
    g_ref[...] = (dinv * hw2).astype(jnp.bfloat16)
    c_ref[...] = (dinv * u_ref[...] + dinv * dinv) * m_ref[...]


def _tc_k2(t_c, u_c, x_c, dinv_c, mask_c, W1, b1, W2):
    col = pl.BlockSpec((_BLK, 1), lambda i: (i, 0))
    full = lambda shape: pl.BlockSpec(shape, lambda i: tuple(0 for _ in shape))
    return pl.pallas_call(
        _tc_k2_body,
        grid=(_GRID,),
        in_specs=[col, col, col, col, col,
                  full((1, _H1)), full((1, _H1)), full((_H1, _H2))],
        out_specs=[pl.BlockSpec((_BLK, _H2), lambda i: (i, 0)), col],
        out_shape=[
            jax.ShapeDtypeStruct((_NPAD, _H2), jnp.bfloat16),
            jax.ShapeDtypeStruct((_NPAD, 1), jnp.float32),
        ],
    )(t_c, u_c, x_c, dinv_c, mask_c, W1, b1, W2)


def _tc_k3_body(acc_ref, g_ref, dinv_ref, c_ref, b2_ref, w3_ref, b3_ref,
                out_ref, racc):
    i = pl.program_id(0)
    a = (acc_ref[0].astype(jnp.float32) + acc_ref[1].astype(jnp.float32)
         + g_ref[...].astype(jnp.float32))
    h2 = dinv_ref[...] * a + b2_ref[...]
    h2 = jnp.where(h2 > 0, h2, 0.1 * h2)
    pr = jnp.sum(c_ref[...] * h2, axis=0, keepdims=True)

    @pl.when(i == 0)
    def _():
        racc[...] = jnp.zeros_like(racc)

    racc[0:1, 0:_H2] += pr
    out_ref[...] = (
        jnp.dot(racc[0:1, 0:_H2], w3_ref[...],
                preferred_element_type=jnp.float32) / _N + b3_ref[...])


def _tc_k3(acc3, g3, dinv_c, c_c, b2, W3, b3):
    col = pl.BlockSpec((_BLK, 1), lambda i: (i, 0))
    full = lambda shape: pl.BlockSpec(shape, lambda i: tuple(0 for _ in shape))
    return pl.pallas_call(
        _tc_k3_body,
        grid=(_GRID,),
        in_specs=[pl.BlockSpec((2, _BLK, _H2), lambda i: (0, i, 0)),
                  pl.BlockSpec((_BLK, _H2), lambda i: (i, 0)), col, col,
                  full((1, _H2)), full((_H2, _H3)), full((1, _H3))],
        out_specs=full((1, _H3)),
        out_shape=jax.ShapeDtypeStruct((1, _H3), jnp.float32),
        scratch_shapes=[pltpu.VMEM((1, _H2), jnp.float32)],
    )(acc3, g3, dinv_c, c_c, b2, W3, b3)


def kernel(x, edge_index, W1, b1, W2, b2, W3, b3):
    pad1 = _N + (jnp.arange(_EPAD - _E, dtype=jnp.int32) % (_NPAD - _N))
    pad = jnp.stack([pad1, pad1])
    edges = jnp.concatenate([edge_index, pad], axis=1)

    xs = jnp.concatenate([x[:, 0], jnp.zeros((_NPAD - _N,), jnp.float32)])
    x_row = xs.reshape(1, _NPAD)
    x_col = xs.reshape(_NPAD, 1)
    mask_c = (jnp.arange(_NPAD) < _N).astype(jnp.float32).reshape(_NPAD, 1)

    deg2 = _sc_deg(edges)
    tab2 = _tc_k1(deg2, x_row)
    tu2 = _sc_tu(edges, tab2)
    dinv_c = tab2[1].reshape(_NPAD, 1)
    t_c = tu2[0].reshape(_NPAD, 1)
    u_c = tu2[1].reshape(_NPAD, 1)
    g3, c_c = _tc_k2(t_c, u_c, x_col, dinv_c, mask_c,
                     W1, b1.reshape(1, _H1), W2)
    acc3 = _sc_acc(edges, g3)
    out = _tc_k3(acc3, g3, dinv_c, c_c, b2.reshape(1, _H2), W3,
                 b3.reshape(1, _H3))
    return out.reshape(_H3)
